# Initial kernel scaffold; baseline (speedup 1.0000x reference)
import jax
import jax.numpy as jnp
from jax.experimental import pallas as pl


def kernel(x, edge_index, edge_attr, batch, Wn, bn, We, be, W1_0, b1_0, W2_0, b2_0, eps_0, g_0, bt_0, W1_1, b1_1, W2_1, b2_1, eps_1, g_1, bt_1, Wc1, bc1, Wc2, bc2):
    raise NotImplementedError("write your pallas kernel here")


# speedup vs baseline: 1.0478x; 1.0478x over previous
import functools
import jax
import jax.numpy as jnp
from jax.experimental import pallas as pl
from jax.experimental.pallas import tpu as pltpu

N = 50000
E = 800000
H = 128
G = 128
BN_ROWS = 2000
NBLK = N // BN_ROWS


def _embed_body(x_ref, wn_ref, bn_ref, out_ref):
    out_ref[...] = x_ref[...] * wn_ref[...] + bn_ref[...]


def _embed(x, Wn, bn):
    return pl.pallas_call(
        _embed_body,
        grid=(NBLK,),
        in_specs=[
            pl.BlockSpec((BN_ROWS, 1), lambda i: (i, 0)),
            pl.BlockSpec((1, H), lambda i: (0, 0)),
            pl.BlockSpec((1, H), lambda i: (0, 0)),
        ],
        out_specs=pl.BlockSpec((BN_ROWS, H), lambda i: (i, 0)),
        out_shape=jax.ShapeDtypeStruct((N, H), jnp.float32),
    )(x, Wn, bn.reshape(1, H))


def _mlp_body(h_ref, aggr_ref, w1_ref, b1_ref, w2_ref, b2_ref, eps_ref,
              z_ref, mom_ref):
    i = pl.program_id(0)
    u = (1.0 + eps_ref[0]) * h_ref[...] + aggr_ref[...]
    t = jnp.maximum(jnp.dot(u, w1_ref[...], preferred_element_type=jnp.float32)
                    + b1_ref[...], 0.0)
    z = jnp.dot(t, w2_ref[...], preferred_element_type=jnp.float32) + b2_ref[...]
    z_ref[...] = z
    mom = jnp.stack([jnp.sum(z, axis=0), jnp.sum(z * z, axis=0)])

    @pl.when(i == 0)
    def _():
        mom_ref[...] = jnp.zeros_like(mom_ref)

    mom_ref[...] += mom


def _mlp(h, aggr, W1, b1, W2, b2, eps):
    return pl.pallas_call(
        _mlp_body,
        grid=(NBLK,),
        in_specs=[
            pl.BlockSpec((BN_ROWS, H), lambda i: (i, 0)),
            pl.BlockSpec((BN_ROWS, H), lambda i: (i, 0)),
            pl.BlockSpec((H, H), lambda i: (0, 0)),
            pl.BlockSpec((1, H), lambda i: (0, 0)),
            pl.BlockSpec((H, H), lambda i: (0, 0)),
            pl.BlockSpec((1, H), lambda i: (0, 0)),
            pl.BlockSpec(memory_space=pltpu.SMEM),
        ],
        out_specs=[
            pl.BlockSpec((BN_ROWS, H), lambda i: (i, 0)),
            pl.BlockSpec((2, H), lambda i: (0, 0)),
        ],
        out_shape=[
            jax.ShapeDtypeStruct((N, H), jnp.float32),
            jax.ShapeDtypeStruct((2, H), jnp.float32),
        ],
    )(h, aggr, W1, b1.reshape(1, H), W2, b2.reshape(1, H),
      eps.reshape(1))


def _bnrelu_body(z_ref, mom_ref, g_ref, bt_ref, out_ref):
    m = mom_ref[0:1, :] / N
    var = mom_ref[1:2, :] / N - m * m
    inv = jax.lax.rsqrt(var + 1e-5)
    scale = g_ref[...] * inv
    shift = bt_ref[...] - m * scale
    out_ref[...] = jnp.maximum(z_ref[...] * scale + shift, 0.0)


def _bnrelu(z, mom, g, bt):
    return pl.pallas_call(
        _bnrelu_body,
        grid=(NBLK,),
        in_specs=[
            pl.BlockSpec((BN_ROWS, H), lambda i: (i, 0)),
            pl.BlockSpec((2, H), lambda i: (0, 0)),
            pl.BlockSpec((1, H), lambda i: (0, 0)),
            pl.BlockSpec((1, H), lambda i: (0, 0)),
        ],
        out_specs=pl.BlockSpec((BN_ROWS, H), lambda i: (i, 0)),
        out_shape=jax.ShapeDtypeStruct((N, H), jnp.float32),
    )(z, mom, g.reshape(1, H), bt.reshape(1, H))


def _pool_head_body(h_ref, batch_ref, wc1_ref, bc1_ref, wc2_ref, bc2_ref,
                    out_ref, sums_ref, cnt_ref):
    i = pl.program_id(0)

    @pl.when(i == 0)
    def _():
        sums_ref[...] = jnp.zeros_like(sums_ref)
        cnt_ref[...] = jnp.zeros_like(cnt_ref)

    b = batch_ref[0, 0, :].reshape(BN_ROWS, 1)
    onehot = (b == jax.lax.broadcasted_iota(jnp.int32, (1, G), 1)
              ).astype(jnp.float32)
    sums_ref[...] += jax.lax.dot_general(
        onehot, h_ref[...], (((0,), (0,)), ((), ())),
        preferred_element_type=jnp.float32)
    cnt_ref[...] += jnp.sum(onehot, axis=0, keepdims=True)

    @pl.when(i == NBLK - 1)
    def _():
        pooled = sums_ref[...] / jnp.maximum(cnt_ref[...], 1.0).reshape(G, 1)
        t = jnp.maximum(
            jnp.dot(pooled, wc1_ref[...], preferred_element_type=jnp.float32)
            + bc1_ref[...], 0.0)
        o = jnp.dot(t, wc2_ref[...], preferred_element_type=jnp.float32) \
            + bc2_ref[...]
        out_ref[...] = jax.nn.sigmoid(o).reshape(1, G)


def _pool_head(h, batch, Wc1, bc1, Wc2, bc2):
    out, _, _ = pl.pallas_call(
        _pool_head_body,
        grid=(NBLK,),
        in_specs=[
            pl.BlockSpec((BN_ROWS, H), lambda i: (i, 0)),
            pl.BlockSpec((1, 1, BN_ROWS), lambda i: (i, 0, 0)),
            pl.BlockSpec((H, H // 2), lambda i: (0, 0)),
            pl.BlockSpec((1, H // 2), lambda i: (0, 0)),
            pl.BlockSpec((H // 2, 1), lambda i: (0, 0)),
            pl.BlockSpec((1, 1), lambda i: (0, 0)),
        ],
        out_specs=[
            pl.BlockSpec((1, G), lambda i: (0, 0)),
            pl.BlockSpec((G, H), lambda i: (0, 0)),
            pl.BlockSpec((1, G), lambda i: (0, 0)),
        ],
        out_shape=[
            jax.ShapeDtypeStruct((1, G), jnp.float32),
            jax.ShapeDtypeStruct((G, H), jnp.float32),
            jax.ShapeDtypeStruct((1, G), jnp.float32),
        ],
    )(h, batch.reshape(NBLK, 1, BN_ROWS), Wc1, bc1.reshape(1, H // 2),
      Wc2, bc2.reshape(1, 1))
    return out.reshape(G)


def kernel(x, edge_index, edge_attr, batch, Wn, bn, We, be, W1_0, b1_0, W2_0,
           b2_0, eps_0, g_0, bt_0, W1_1, b1_1, W2_1, b2_1, eps_1, g_1, bt_1,
           Wc1, bc1, Wc2, bc2):
    src, dst = edge_index[0], edge_index[1]
    h = _embed(x, Wn, bn)
    ea = edge_attr * We.reshape(1, H) + be.reshape(1, H)
    layers = ((W1_0, b1_0, W2_0, b2_0, eps_0, g_0, bt_0),
              (W1_1, b1_1, W2_1, b2_1, eps_1, g_1, bt_1))
    for (W1, b1, W2, b2, eps, g, bt) in layers:
        msg = jax.nn.relu(h[src] + ea)
        aggr = jnp.zeros_like(h).at[dst].add(msg)
        z, mom = _mlp(h, aggr, W1, b1, W2, b2, eps)
        h = _bnrelu(z, mom, g, bt)
    return _pool_head(h, batch, Wc1, bc1, Wc2, bc2)


# speedup vs baseline: 1.1222x; 1.0710x over previous
import functools
import jax
import jax.numpy as jnp
from jax import lax
from jax.experimental import pallas as pl
from jax.experimental.pallas import tpu as pltpu
from jax.experimental.pallas import tpu_sc as plsc

N = 50000
E = 800000
H = 128
G = 128
BN_ROWS = 2000
NBLK = N // BN_ROWS

RCHUNK = 12800
NPASS = 2
NPAD = NPASS * 2 * RCHUNK
ET = E // 16
SB = 2000
NSB = ET // SB
GROUPS = SB // 16
BE = 80
PCAP = SB + BE
ZR = 16
RSUB = RCHUNK // 16


def _embed_body(x_ref, wn_ref, bn_ref, be_ref, out_ref, outg_ref):
    h = x_ref[...] * wn_ref[...] + bn_ref[...]
    out_ref[...] = h
    outg_ref[...] = h + be_ref[...]


def _embed(x, Wn, bn, be):
    return pl.pallas_call(
        _embed_body,
        grid=(NBLK,),
        in_specs=[
            pl.BlockSpec((BN_ROWS, 1), lambda i: (i, 0)),
            pl.BlockSpec((1, H), lambda i: (0, 0)),
            pl.BlockSpec((1, H), lambda i: (0, 0)),
            pl.BlockSpec((1, H), lambda i: (0, 0)),
        ],
        out_specs=[
            pl.BlockSpec((BN_ROWS, H), lambda i: (i, 0)),
            pl.BlockSpec((BN_ROWS, H), lambda i: (i, 0)),
        ],
        out_shape=[
            jax.ShapeDtypeStruct((N, H), jnp.float32),
            jax.ShapeDtypeStruct((N, H), jnp.float32),
        ],
    )(x, Wn, bn.reshape(1, H), be.reshape(1, H))


ERB = 5000


def _attrx_body(a_ref, out_ref):
    a = a_ref[...]
    parts = [jnp.broadcast_to(a[:, k:k + 1], (ERB, 16)) for k in range(8)]
    out_ref[...] = jnp.concatenate(parts, axis=1)


def _attr_expand(attr):
    out = pl.pallas_call(
        _attrx_body,
        grid=(E // 8 // ERB,),
        in_specs=[pl.BlockSpec((ERB, 8), lambda i: (i, 0))],
        out_specs=pl.BlockSpec((ERB, 128), lambda i: (i, 0)),
        out_shape=jax.ShapeDtypeStruct((E // 8, 128), jnp.float32),
    )(attr.reshape(E // 8, 8))
    return out.reshape(E * 16)


def _mp_body(hg, src_h, dst_h, attrx_h, wv_h, out,
             accum, wv, sb_src, sb_dst, ax, sidx, rows, zb,
             sem):
    cid = lax.axis_index("c")
    sid = lax.axis_index("s")
    ebase0 = sid * ET

    def zrow(r, _):
        for f in range(H // 16):
            zb[r, pl.ds(16 * f, 16)] = jnp.zeros((16,), jnp.float32)
        return 0
    lax.fori_loop(0, ZR, zrow, 0)
    pltpu.sync_copy(wv_h, wv)
    wvs = tuple(wv[pl.ds(16 * f, 16)] for f in range(H // 16))

    for p in range(NPASS):
        lo = (2 * p + cid) * RCHUNK
        hi = lo + RCHUNK

        def zc(i, _):
            pltpu.sync_copy(zb, accum.at[pl.ds(sid * RSUB + i * ZR, ZR)])
            return 0
        lax.fori_loop(0, RSUB // ZR, zc, 0)

        @pl.when(sid == 0)
        def _():
            pltpu.sync_copy(zb.at[pl.ds(0, 8)], accum.at[pl.ds(RCHUNK, 8)])

        plsc.subcore_barrier()

        def sb_body(sb, wvs):
            ebase = ebase0 + sb * SB
            pltpu.sync_copy(src_h.at[pl.ds(ebase, SB)], sb_src)
            pltpu.sync_copy(dst_h.at[pl.ds(ebase, SB)], sb_dst)

            def fire(b, wvs):
                boff = b * BE
                for j in range(BE // 16):
                    d = sb_dst[pl.ds(boff + 16 * j, 16)]
                    mi = jnp.clip(jnp.minimum(d - lo + 1, hi - d), 0, 1)
                    sidx[pl.ds(16 * j, 16)] = (mi * (d - lo)
                                               + (1 - mi) * RCHUNK)
                pltpu.sync_copy(
                    attrx_h.at[pl.ds((ebase + boff) * 16, BE * 16)], ax)
                pltpu.async_copy(hg.at[sb_src.at[pl.ds(boff, BE)]],
                                 rows, sem).wait()

                def edge(e, wvs):
                    sp = ax[pl.ds(e * 16, 16)]
                    for f in range(H // 16):
                        r = rows[e, pl.ds(16 * f, 16)]
                        rows[e, pl.ds(16 * f, 16)] = jnp.maximum(
                            sp * wvs[f] + r, 0.0)
                    return wvs
                wvs = lax.fori_loop(0, BE, edge, wvs)
                pltpu.sync_copy(rows, accum.at[sidx], add=True)
                return wvs

            return lax.fori_loop(0, SB // BE, fire, wvs)

        wvs = lax.fori_loop(0, NSB, sb_body, wvs)
        plsc.subcore_barrier()
        pltpu.sync_copy(accum.at[pl.ds(sid * RSUB, RSUB)],
                        out.at[pl.ds(lo + sid * RSUB, RSUB)])
        plsc.subcore_barrier()


@functools.cache
def _mp_call():
    return pl.kernel(
        _mp_body,
        out_type=jax.ShapeDtypeStruct((NPAD, H), jnp.float32),
        mesh=plsc.VectorSubcoreMesh(core_axis_name="c", subcore_axis_name="s",
                                    num_cores=2, num_subcores=16),
        scratch_types=[
            pltpu.VMEM_SHARED((RCHUNK + 8, H), jnp.float32),
            pltpu.VMEM((H,), jnp.float32),
            pltpu.VMEM((SB,), jnp.int32),
            pltpu.VMEM((SB,), jnp.int32),
            pltpu.VMEM((BE * 16,), jnp.float32),
            pltpu.VMEM((BE,), jnp.int32),
            pltpu.VMEM((BE, H), jnp.float32),
            pltpu.VMEM((ZR, H), jnp.float32),
            pltpu.SemaphoreType.DMA,
        ],
    )


def _message_pass(hg, src, dst, attrx, wvec):
    return _mp_call()(hg, src, dst, attrx, wvec)


def _mlp_body(h_ref, aggr_ref, w1_ref, b1_ref, w2_ref, b2_ref, eps_ref,
              z_ref, mom_ref):
    i = pl.program_id(0)
    u = (1.0 + eps_ref[0]) * h_ref[...] + aggr_ref[...]
    t = jnp.maximum(jnp.dot(u, w1_ref[...], preferred_element_type=jnp.float32)
                    + b1_ref[...], 0.0)
    z = jnp.dot(t, w2_ref[...], preferred_element_type=jnp.float32) + b2_ref[...]
    z_ref[...] = z
    mom = jnp.stack([jnp.sum(z, axis=0), jnp.sum(z * z, axis=0)])

    @pl.when(i == 0)
    def _():
        mom_ref[...] = jnp.zeros_like(mom_ref)

    mom_ref[...] += mom


def _mlp(h, aggr, W1, b1, W2, b2, eps):
    return pl.pallas_call(
        _mlp_body,
        grid=(NBLK,),
        in_specs=[
            pl.BlockSpec((BN_ROWS, H), lambda i: (i, 0)),
            pl.BlockSpec((BN_ROWS, H), lambda i: (i, 0)),
            pl.BlockSpec((H, H), lambda i: (0, 0)),
            pl.BlockSpec((1, H), lambda i: (0, 0)),
            pl.BlockSpec((H, H), lambda i: (0, 0)),
            pl.BlockSpec((1, H), lambda i: (0, 0)),
            pl.BlockSpec(memory_space=pltpu.SMEM),
        ],
        out_specs=[
            pl.BlockSpec((BN_ROWS, H), lambda i: (i, 0)),
            pl.BlockSpec((2, H), lambda i: (0, 0)),
        ],
        out_shape=[
            jax.ShapeDtypeStruct((N, H), jnp.float32),
            jax.ShapeDtypeStruct((2, H), jnp.float32),
        ],
    )(h, aggr, W1, b1.reshape(1, H), W2, b2.reshape(1, H),
      eps.reshape(1))


def _bnrelu_body(z_ref, mom_ref, g_ref, bt_ref, be_ref, out_ref, outg_ref):
    m = mom_ref[0:1, :] / N
    var = mom_ref[1:2, :] / N - m * m
    inv = jax.lax.rsqrt(var + 1e-5)
    scale = g_ref[...] * inv
    shift = bt_ref[...] - m * scale
    h = jnp.maximum(z_ref[...] * scale + shift, 0.0)
    out_ref[...] = h
    outg_ref[...] = h + be_ref[...]


def _bnrelu(z, mom, g, bt, be):
    return pl.pallas_call(
        _bnrelu_body,
        grid=(NBLK,),
        in_specs=[
            pl.BlockSpec((BN_ROWS, H), lambda i: (i, 0)),
            pl.BlockSpec((2, H), lambda i: (0, 0)),
            pl.BlockSpec((1, H), lambda i: (0, 0)),
            pl.BlockSpec((1, H), lambda i: (0, 0)),
            pl.BlockSpec((1, H), lambda i: (0, 0)),
        ],
        out_specs=[
            pl.BlockSpec((BN_ROWS, H), lambda i: (i, 0)),
            pl.BlockSpec((BN_ROWS, H), lambda i: (i, 0)),
        ],
        out_shape=[
            jax.ShapeDtypeStruct((N, H), jnp.float32),
            jax.ShapeDtypeStruct((N, H), jnp.float32),
        ],
    )(z, mom, g.reshape(1, H), bt.reshape(1, H), be.reshape(1, H))


def _bnrelu_last_body(z_ref, mom_ref, g_ref, bt_ref, out_ref):
    m = mom_ref[0:1, :] / N
    var = mom_ref[1:2, :] / N - m * m
    inv = jax.lax.rsqrt(var + 1e-5)
    scale = g_ref[...] * inv
    shift = bt_ref[...] - m * scale
    out_ref[...] = jnp.maximum(z_ref[...] * scale + shift, 0.0)


def _bnrelu_last(z, mom, g, bt):
    return pl.pallas_call(
        _bnrelu_last_body,
        grid=(NBLK,),
        in_specs=[
            pl.BlockSpec((BN_ROWS, H), lambda i: (i, 0)),
            pl.BlockSpec((2, H), lambda i: (0, 0)),
            pl.BlockSpec((1, H), lambda i: (0, 0)),
            pl.BlockSpec((1, H), lambda i: (0, 0)),
        ],
        out_specs=pl.BlockSpec((BN_ROWS, H), lambda i: (i, 0)),
        out_shape=jax.ShapeDtypeStruct((N, H), jnp.float32),
    )(z, mom, g.reshape(1, H), bt.reshape(1, H))


def _pool_head_body(h_ref, batch_ref, wc1_ref, bc1_ref, wc2_ref, bc2_ref,
                    out_ref, sums_ref, cnt_ref):
    i = pl.program_id(0)

    @pl.when(i == 0)
    def _():
        sums_ref[...] = jnp.zeros_like(sums_ref)
        cnt_ref[...] = jnp.zeros_like(cnt_ref)

    b = batch_ref[0, 0, :].reshape(BN_ROWS, 1)
    onehot = (b == jax.lax.broadcasted_iota(jnp.int32, (1, G), 1)
              ).astype(jnp.float32)
    sums_ref[...] += jax.lax.dot_general(
        onehot, h_ref[...], (((0,), (0,)), ((), ())),
        preferred_element_type=jnp.float32)
    cnt_ref[...] += jnp.sum(onehot, axis=0, keepdims=True)

    @pl.when(i == NBLK - 1)
    def _():
        pooled = sums_ref[...] / jnp.maximum(cnt_ref[...], 1.0).reshape(G, 1)
        t = jnp.maximum(
            jnp.dot(pooled, wc1_ref[...], preferred_element_type=jnp.float32)
            + bc1_ref[...], 0.0)
        o = jnp.dot(t, wc2_ref[...], preferred_element_type=jnp.float32) \
            + bc2_ref[...]
        out_ref[...] = jax.nn.sigmoid(o).reshape(1, G)


def _pool_head(h, batch, Wc1, bc1, Wc2, bc2):
    out, _, _ = pl.pallas_call(
        _pool_head_body,
        grid=(NBLK,),
        in_specs=[
            pl.BlockSpec((BN_ROWS, H), lambda i: (i, 0)),
            pl.BlockSpec((1, 1, BN_ROWS), lambda i: (i, 0, 0)),
            pl.BlockSpec((H, H // 2), lambda i: (0, 0)),
            pl.BlockSpec((1, H // 2), lambda i: (0, 0)),
            pl.BlockSpec((H // 2, 1), lambda i: (0, 0)),
            pl.BlockSpec((1, 1), lambda i: (0, 0)),
        ],
        out_specs=[
            pl.BlockSpec((1, G), lambda i: (0, 0)),
            pl.BlockSpec((G, H), lambda i: (0, 0)),
            pl.BlockSpec((1, G), lambda i: (0, 0)),
        ],
        out_shape=[
            jax.ShapeDtypeStruct((1, G), jnp.float32),
            jax.ShapeDtypeStruct((G, H), jnp.float32),
            jax.ShapeDtypeStruct((1, G), jnp.float32),
        ],
    )(h, batch.reshape(NBLK, 1, BN_ROWS), Wc1, bc1.reshape(1, H // 2),
      Wc2, bc2.reshape(1, 1))
    return out.reshape(G)


def kernel(x, edge_index, edge_attr, batch, Wn, bn, We, be, W1_0, b1_0, W2_0,
           b2_0, eps_0, g_0, bt_0, W1_1, b1_1, W2_1, b2_1, eps_1, g_1, bt_1,
           Wc1, bc1, Wc2, bc2):
    src, dst = edge_index[0], edge_index[1]
    attrx = _attr_expand(edge_attr.reshape(E))
    wvec = We.reshape(H)
    h, hg = _embed(x, Wn, bn, be)
    layers = ((W1_0, b1_0, W2_0, b2_0, eps_0, g_0, bt_0),
              (W1_1, b1_1, W2_1, b2_1, eps_1, g_1, bt_1))
    for li, (W1, b1, W2, b2, eps, g, bt) in enumerate(layers):
        aggr = _message_pass(hg, src, dst, attrx, wvec)
        z, mom = _mlp(h, aggr, W1, b1, W2, b2, eps)
        if li == 0:
            h, hg = _bnrelu(z, mom, g, bt, be)
        else:
            h = _bnrelu_last(z, mom, g, bt)
    return _pool_head(h, batch, Wc1, bc1, Wc2, bc2)


# speedup vs baseline: 1.9881x; 1.7716x over previous
import functools
import jax
import jax.numpy as jnp
from jax import lax
from jax.experimental import pallas as pl
from jax.experimental.pallas import tpu as pltpu
from jax.experimental.pallas import tpu_sc as plsc

N = 50000
E = 800000
H = 128
G = 128
BN_ROWS = 2000
NBLK = N // BN_ROWS

RCHUNK = 12800
NPASS = 2
NPAD = NPASS * 2 * RCHUNK
ET = E // 16
SB = 2000
NSB = ET // SB
GROUPS = SB // 16
BE = 80
PCAP = SB + BE
ZR = 8
RSUB = RCHUNK // 16


def _embed_body(x_ref, wn_ref, bn_ref, be_ref, out_ref, outg_ref):
    h = x_ref[...] * wn_ref[...] + bn_ref[...]
    out_ref[...] = h
    outg_ref[...] = h + be_ref[...]


def _embed(x, Wn, bn, be):
    return pl.pallas_call(
        _embed_body,
        grid=(NBLK,),
        in_specs=[
            pl.BlockSpec((BN_ROWS, 1), lambda i: (i, 0)),
            pl.BlockSpec((1, H), lambda i: (0, 0)),
            pl.BlockSpec((1, H), lambda i: (0, 0)),
            pl.BlockSpec((1, H), lambda i: (0, 0)),
        ],
        out_specs=[
            pl.BlockSpec((BN_ROWS, H), lambda i: (i, 0)),
            pl.BlockSpec((BN_ROWS, H), lambda i: (i, 0)),
        ],
        out_shape=[
            jax.ShapeDtypeStruct((N, H), jnp.float32),
            jax.ShapeDtypeStruct((N, H), jnp.float32),
        ],
    )(x, Wn, bn.reshape(1, H), be.reshape(1, H))


ERB = 5000


def _attrx_body(a_ref, out_ref):
    a = a_ref[...]
    parts = [jnp.broadcast_to(a[:, k:k + 1], (ERB, 16)) for k in range(8)]
    out_ref[...] = jnp.concatenate(parts, axis=1)


def _attr_expand(attr):
    out = pl.pallas_call(
        _attrx_body,
        grid=(E // 8 // ERB,),
        in_specs=[pl.BlockSpec((ERB, 8), lambda i: (i, 0))],
        out_specs=pl.BlockSpec((ERB, 128), lambda i: (i, 0)),
        out_shape=jax.ShapeDtypeStruct((E // 8, 128), jnp.float32),
    )(attr.reshape(E // 8, 8))
    return out.reshape(E * 16)


NF = SB // BE


def _mp_body(hg, src_h, dst_h, attrx_h, wv_h, out,
             accum, wv, sb_src, sb_dst, ax0, ax1, sidx, rows0, rows1, zb,
             semr0, semr1, sema0, sema1):
    cid = lax.axis_index("c")
    sid = lax.axis_index("s")
    ebase0 = sid * ET

    def zrow(r, _):
        for f in range(H // 16):
            zb[r, pl.ds(16 * f, 16)] = jnp.zeros((16,), jnp.float32)
        return 0
    lax.fori_loop(0, ZR, zrow, 0)
    pltpu.sync_copy(wv_h, wv)

    for p in range(NPASS):
        lo = (2 * p + cid) * RCHUNK
        hi = lo + RCHUNK

        def zc(i, _):
            pltpu.sync_copy(zb, accum.at[pl.ds(sid * RSUB + i * ZR, ZR)])
            return 0
        lax.fori_loop(0, RSUB // ZR, zc, 0)

        @pl.when(sid == 0)
        def _():
            pltpu.sync_copy(zb.at[pl.ds(0, 8)], accum.at[pl.ds(RCHUNK, 8)])

        plsc.subcore_barrier()

        def sb_body(sb, _):
            ebase = ebase0 + sb * SB
            pltpu.sync_copy(src_h.at[pl.ds(ebase, SB)], sb_src)
            pltpu.sync_copy(dst_h.at[pl.ds(ebase, SB)], sb_dst)

            def start_fire(b, rows_b, ax_b, semr, sema):
                boff = b * BE
                pltpu.async_copy(
                    attrx_h.at[pl.ds((ebase + boff) * 16, BE * 16)],
                    ax_b, sema)
                pltpu.async_copy(hg.at[sb_src.at[pl.ds(boff, BE)]],
                                 rows_b, semr)

            def finish_fire(b, rows_b, ax_b, semr, sema):
                boff = b * BE
                for j in range(BE // 16):
                    d = sb_dst[pl.ds(boff + 16 * j, 16)]
                    mi = jnp.clip(jnp.minimum(d - lo + 1, hi - d), 0, 1)
                    sidx[pl.ds(16 * j, 16)] = (mi * (d - lo)
                                               + (1 - mi) * RCHUNK)
                pltpu.make_async_copy(
                    attrx_h.at[pl.ds((ebase + boff) * 16, BE * 16)],
                    ax_b, sema).wait()
                pltpu.make_async_copy(hg.at[sb_src.at[pl.ds(boff, BE)]],
                                      rows_b, semr).wait()
                wvs = tuple(wv[pl.ds(16 * f, 16)] for f in range(H // 16))

                def egroup(g, _):
                    base = g * 16
                    for k in range(16):
                        e = base + k
                        sp = ax_b[pl.ds(e * 16, 16)]
                        for f in range(H // 16):
                            r = rows_b[e, pl.ds(16 * f, 16)]
                            rows_b[e, pl.ds(16 * f, 16)] = jnp.maximum(
                                sp * wvs[f] + r, 0.0)
                    return 0
                lax.fori_loop(0, BE // 16, egroup, 0)
                pltpu.sync_copy(rows_b, accum.at[sidx], add=True)

            start_fire(0, rows0, ax0, semr0, sema0)

            def fire(b, _):
                nb = b + 1

                @pl.when(b % 2 == 0)
                def _():
                    @pl.when(nb < NF)
                    def _():
                        start_fire(nb, rows1, ax1, semr1, sema1)
                    finish_fire(b, rows0, ax0, semr0, sema0)

                @pl.when(b % 2 == 1)
                def _():
                    @pl.when(nb < NF)
                    def _():
                        start_fire(nb, rows0, ax0, semr0, sema0)
                    finish_fire(b, rows1, ax1, semr1, sema1)
                return 0

            return lax.fori_loop(0, NF, fire, 0)

        lax.fori_loop(0, NSB, sb_body, 0)
        plsc.subcore_barrier()
        pltpu.sync_copy(accum.at[pl.ds(sid * RSUB, RSUB)],
                        out.at[pl.ds(lo + sid * RSUB, RSUB)])
        plsc.subcore_barrier()


@functools.cache
def _mp_call():
    return pl.kernel(
        _mp_body,
        out_type=jax.ShapeDtypeStruct((NPAD, H), jnp.float32),
        mesh=plsc.VectorSubcoreMesh(core_axis_name="c", subcore_axis_name="s",
                                    num_cores=2, num_subcores=16),
        scratch_types=[
            pltpu.VMEM_SHARED((RCHUNK + 8, H), jnp.float32),
            pltpu.VMEM((H,), jnp.float32),
            pltpu.VMEM((SB,), jnp.int32),
            pltpu.VMEM((SB,), jnp.int32),
            pltpu.VMEM((BE * 16,), jnp.float32),
            pltpu.VMEM((BE * 16,), jnp.float32),
            pltpu.VMEM((BE,), jnp.int32),
            pltpu.VMEM((BE, H), jnp.float32),
            pltpu.VMEM((BE, H), jnp.float32),
            pltpu.VMEM((ZR, H), jnp.float32),
            pltpu.SemaphoreType.DMA,
            pltpu.SemaphoreType.DMA,
            pltpu.SemaphoreType.DMA,
            pltpu.SemaphoreType.DMA,
        ],
    )


def _message_pass(hg, src, dst, attrx, wvec):
    return _mp_call()(hg, src, dst, attrx, wvec)


def _mlp_body(h_ref, aggr_ref, w1_ref, b1_ref, w2_ref, b2_ref, eps_ref,
              z_ref, mom_ref):
    i = pl.program_id(0)
    u = (1.0 + eps_ref[0]) * h_ref[...] + aggr_ref[...]
    t = jnp.maximum(jnp.dot(u, w1_ref[...], preferred_element_type=jnp.float32)
                    + b1_ref[...], 0.0)
    z = jnp.dot(t, w2_ref[...], preferred_element_type=jnp.float32) + b2_ref[...]
    z_ref[...] = z
    mom = jnp.stack([jnp.sum(z, axis=0), jnp.sum(z * z, axis=0)])

    @pl.when(i == 0)
    def _():
        mom_ref[...] = jnp.zeros_like(mom_ref)

    mom_ref[...] += mom


def _mlp(h, aggr, W1, b1, W2, b2, eps):
    return pl.pallas_call(
        _mlp_body,
        grid=(NBLK,),
        in_specs=[
            pl.BlockSpec((BN_ROWS, H), lambda i: (i, 0)),
            pl.BlockSpec((BN_ROWS, H), lambda i: (i, 0)),
            pl.BlockSpec((H, H), lambda i: (0, 0)),
            pl.BlockSpec((1, H), lambda i: (0, 0)),
            pl.BlockSpec((H, H), lambda i: (0, 0)),
            pl.BlockSpec((1, H), lambda i: (0, 0)),
            pl.BlockSpec(memory_space=pltpu.SMEM),
        ],
        out_specs=[
            pl.BlockSpec((BN_ROWS, H), lambda i: (i, 0)),
            pl.BlockSpec((2, H), lambda i: (0, 0)),
        ],
        out_shape=[
            jax.ShapeDtypeStruct((N, H), jnp.float32),
            jax.ShapeDtypeStruct((2, H), jnp.float32),
        ],
    )(h, aggr, W1, b1.reshape(1, H), W2, b2.reshape(1, H),
      eps.reshape(1))


def _bnrelu_body(z_ref, mom_ref, g_ref, bt_ref, be_ref, out_ref, outg_ref):
    m = mom_ref[0:1, :] / N
    var = mom_ref[1:2, :] / N - m * m
    inv = jax.lax.rsqrt(var + 1e-5)
    scale = g_ref[...] * inv
    shift = bt_ref[...] - m * scale
    h = jnp.maximum(z_ref[...] * scale + shift, 0.0)
    out_ref[...] = h
    outg_ref[...] = h + be_ref[...]


def _bnrelu(z, mom, g, bt, be):
    return pl.pallas_call(
        _bnrelu_body,
        grid=(NBLK,),
        in_specs=[
            pl.BlockSpec((BN_ROWS, H), lambda i: (i, 0)),
            pl.BlockSpec((2, H), lambda i: (0, 0)),
            pl.BlockSpec((1, H), lambda i: (0, 0)),
            pl.BlockSpec((1, H), lambda i: (0, 0)),
            pl.BlockSpec((1, H), lambda i: (0, 0)),
        ],
        out_specs=[
            pl.BlockSpec((BN_ROWS, H), lambda i: (i, 0)),
            pl.BlockSpec((BN_ROWS, H), lambda i: (i, 0)),
        ],
        out_shape=[
            jax.ShapeDtypeStruct((N, H), jnp.float32),
            jax.ShapeDtypeStruct((N, H), jnp.float32),
        ],
    )(z, mom, g.reshape(1, H), bt.reshape(1, H), be.reshape(1, H))


def _bnrelu_last_body(z_ref, mom_ref, g_ref, bt_ref, out_ref):
    m = mom_ref[0:1, :] / N
    var = mom_ref[1:2, :] / N - m * m
    inv = jax.lax.rsqrt(var + 1e-5)
    scale = g_ref[...] * inv
    shift = bt_ref[...] - m * scale
    out_ref[...] = jnp.maximum(z_ref[...] * scale + shift, 0.0)


def _bnrelu_last(z, mom, g, bt):
    return pl.pallas_call(
        _bnrelu_last_body,
        grid=(NBLK,),
        in_specs=[
            pl.BlockSpec((BN_ROWS, H), lambda i: (i, 0)),
            pl.BlockSpec((2, H), lambda i: (0, 0)),
            pl.BlockSpec((1, H), lambda i: (0, 0)),
            pl.BlockSpec((1, H), lambda i: (0, 0)),
        ],
        out_specs=pl.BlockSpec((BN_ROWS, H), lambda i: (i, 0)),
        out_shape=jax.ShapeDtypeStruct((N, H), jnp.float32),
    )(z, mom, g.reshape(1, H), bt.reshape(1, H))


def _pool_head_body(h_ref, batch_ref, wc1_ref, bc1_ref, wc2_ref, bc2_ref,
                    out_ref, sums_ref, cnt_ref):
    i = pl.program_id(0)

    @pl.when(i == 0)
    def _():
        sums_ref[...] = jnp.zeros_like(sums_ref)
        cnt_ref[...] = jnp.zeros_like(cnt_ref)

    b = batch_ref[0, 0, :].reshape(BN_ROWS, 1)
    onehot = (b == jax.lax.broadcasted_iota(jnp.int32, (1, G), 1)
              ).astype(jnp.float32)
    sums_ref[...] += jax.lax.dot_general(
        onehot, h_ref[...], (((0,), (0,)), ((), ())),
        preferred_element_type=jnp.float32)
    cnt_ref[...] += jnp.sum(onehot, axis=0, keepdims=True)

    @pl.when(i == NBLK - 1)
    def _():
        pooled = sums_ref[...] / jnp.maximum(cnt_ref[...], 1.0).reshape(G, 1)
        t = jnp.maximum(
            jnp.dot(pooled, wc1_ref[...], preferred_element_type=jnp.float32)
            + bc1_ref[...], 0.0)
        o = jnp.dot(t, wc2_ref[...], preferred_element_type=jnp.float32) \
            + bc2_ref[...]
        out_ref[...] = jax.nn.sigmoid(o).reshape(1, G)


def _pool_head(h, batch, Wc1, bc1, Wc2, bc2):
    out, _, _ = pl.pallas_call(
        _pool_head_body,
        grid=(NBLK,),
        in_specs=[
            pl.BlockSpec((BN_ROWS, H), lambda i: (i, 0)),
            pl.BlockSpec((1, 1, BN_ROWS), lambda i: (i, 0, 0)),
            pl.BlockSpec((H, H // 2), lambda i: (0, 0)),
            pl.BlockSpec((1, H // 2), lambda i: (0, 0)),
            pl.BlockSpec((H // 2, 1), lambda i: (0, 0)),
            pl.BlockSpec((1, 1), lambda i: (0, 0)),
        ],
        out_specs=[
            pl.BlockSpec((1, G), lambda i: (0, 0)),
            pl.BlockSpec((G, H), lambda i: (0, 0)),
            pl.BlockSpec((1, G), lambda i: (0, 0)),
        ],
        out_shape=[
            jax.ShapeDtypeStruct((1, G), jnp.float32),
            jax.ShapeDtypeStruct((G, H), jnp.float32),
            jax.ShapeDtypeStruct((1, G), jnp.float32),
        ],
    )(h, batch.reshape(NBLK, 1, BN_ROWS), Wc1, bc1.reshape(1, H // 2),
      Wc2, bc2.reshape(1, 1))
    return out.reshape(G)


def kernel(x, edge_index, edge_attr, batch, Wn, bn, We, be, W1_0, b1_0, W2_0,
           b2_0, eps_0, g_0, bt_0, W1_1, b1_1, W2_1, b2_1, eps_1, g_1, bt_1,
           Wc1, bc1, Wc2, bc2):
    src, dst = edge_index[0], edge_index[1]
    attrx = _attr_expand(edge_attr.reshape(E))
    wvec = We.reshape(H)
    h, hg = _embed(x, Wn, bn, be)
    layers = ((W1_0, b1_0, W2_0, b2_0, eps_0, g_0, bt_0),
              (W1_1, b1_1, W2_1, b2_1, eps_1, g_1, bt_1))
    for li, (W1, b1, W2, b2, eps, g, bt) in enumerate(layers):
        aggr = _message_pass(hg, src, dst, attrx, wvec)
        z, mom = _mlp(h, aggr, W1, b1, W2, b2, eps)
        if li == 0:
            h, hg = _bnrelu(z, mom, g, bt, be)
        else:
            h = _bnrelu_last(z, mom, g, bt)
    return _pool_head(h, batch, Wc1, bc1, Wc2, bc2)


# speedup vs baseline: 1.9984x; 1.0052x over previous
import functools
import jax
import jax.numpy as jnp
from jax import lax
from jax.experimental import pallas as pl
from jax.experimental.pallas import tpu as pltpu
from jax.experimental.pallas import tpu_sc as plsc

N = 50000
E = 800000
H = 128
G = 128
BN_ROWS = 2000
NBLK = N // BN_ROWS

RCHUNK = 12800
NPASS = 2
NPAD = NPASS * 2 * RCHUNK
ET = E // 16
SB = 2000
NSB = ET // SB
GROUPS = SB // 16
BE = 80
PCAP = SB + BE
ZR = 8
RSUB = RCHUNK // 16


def _embed_body(x_ref, wn_ref, bn_ref, be_ref, out_ref, outg_ref):
    h = x_ref[...] * wn_ref[...] + bn_ref[...]
    out_ref[...] = h
    outg_ref[...] = h + be_ref[...]


def _embed(x, Wn, bn, be):
    return pl.pallas_call(
        _embed_body,
        grid=(NBLK,),
        in_specs=[
            pl.BlockSpec((BN_ROWS, 1), lambda i: (i, 0)),
            pl.BlockSpec((1, H), lambda i: (0, 0)),
            pl.BlockSpec((1, H), lambda i: (0, 0)),
            pl.BlockSpec((1, H), lambda i: (0, 0)),
        ],
        out_specs=[
            pl.BlockSpec((BN_ROWS, H), lambda i: (i, 0)),
            pl.BlockSpec((BN_ROWS, H), lambda i: (i, 0)),
        ],
        out_shape=[
            jax.ShapeDtypeStruct((N, H), jnp.float32),
            jax.ShapeDtypeStruct((N, H), jnp.float32),
        ],
    )(x, Wn, bn.reshape(1, H), be.reshape(1, H))


ERB = 5000


def _attrx_body(a_ref, out_ref):
    a = a_ref[...]
    parts = [jnp.broadcast_to(a[:, k:k + 1], (ERB, 16)) for k in range(8)]
    out_ref[...] = jnp.concatenate(parts, axis=1)


def _attr_expand(attr):
    out = pl.pallas_call(
        _attrx_body,
        grid=(E // 8 // ERB,),
        in_specs=[pl.BlockSpec((ERB, 8), lambda i: (i, 0))],
        out_specs=pl.BlockSpec((ERB, 128), lambda i: (i, 0)),
        out_shape=jax.ShapeDtypeStruct((E // 8, 128), jnp.float32),
    )(attr.reshape(E // 8, 8))
    return out.reshape(E * 16)


NF = SB // BE


def _mp_body(hg, src_h, dst_h, attrx_h, wv_h, out,
             accum, wv, sb_src, sb_dst, ax0, ax1, sidx0, sidx1, rows0, rows1,
             zb, semr0, semr1, sema0, sema1, sems0, sems1):
    cid = lax.axis_index("c")
    sid = lax.axis_index("s")
    ebase0 = sid * ET

    def zrow(r, _):
        for f in range(H // 16):
            zb[r, pl.ds(16 * f, 16)] = jnp.zeros((16,), jnp.float32)
        return 0
    lax.fori_loop(0, ZR, zrow, 0)
    pltpu.sync_copy(wv_h, wv)

    for p in range(NPASS):
        lo = (2 * p + cid) * RCHUNK
        hi = lo + RCHUNK

        def zc(i, _):
            pltpu.sync_copy(zb, accum.at[pl.ds(sid * RSUB + i * ZR, ZR)])
            return 0
        lax.fori_loop(0, RSUB // ZR, zc, 0)

        @pl.when(sid == 0)
        def _():
            pltpu.sync_copy(zb.at[pl.ds(0, 8)], accum.at[pl.ds(RCHUNK, 8)])

        plsc.subcore_barrier()

        def sb_body(sb, _):
            ebase = ebase0 + sb * SB
            pltpu.sync_copy(src_h.at[pl.ds(ebase, SB)], sb_src)
            pltpu.sync_copy(dst_h.at[pl.ds(ebase, SB)], sb_dst)

            def start_fire(b, rows_b, ax_b, semr, sema):
                boff = b * BE
                pltpu.async_copy(
                    attrx_h.at[pl.ds((ebase + boff) * 16, BE * 16)],
                    ax_b, sema)
                pltpu.async_copy(hg.at[sb_src.at[pl.ds(boff, BE)]],
                                 rows_b, semr)

            def finish_fire(b, rows_b, ax_b, sidx_b, semr, sema, sems):
                boff = b * BE
                for j in range(BE // 16):
                    d = sb_dst[pl.ds(boff + 16 * j, 16)]
                    mi = jnp.clip(jnp.minimum(d - lo + 1, hi - d), 0, 1)
                    sidx_b[pl.ds(16 * j, 16)] = (mi * (d - lo)
                                                 + (1 - mi) * RCHUNK)
                pltpu.make_async_copy(
                    attrx_h.at[pl.ds((ebase + boff) * 16, BE * 16)],
                    ax_b, sema).wait()
                pltpu.make_async_copy(hg.at[sb_src.at[pl.ds(boff, BE)]],
                                      rows_b, semr).wait()
                wvs = tuple(wv[pl.ds(16 * f, 16)] for f in range(H // 16))

                def egroup(g, _):
                    base = g * 16
                    for k in range(16):
                        e = base + k
                        sp = ax_b[pl.ds(e * 16, 16)]
                        for f in range(H // 16):
                            r = rows_b[e, pl.ds(16 * f, 16)]
                            rows_b[e, pl.ds(16 * f, 16)] = jnp.maximum(
                                sp * wvs[f] + r, 0.0)
                    return 0
                lax.fori_loop(0, BE // 16, egroup, 0)
                pltpu.async_copy(rows_b, accum.at[sidx_b], sems, add=True)

            def wait_scatter(rows_b, sidx_b, sems):
                pltpu.make_async_copy(rows_b, accum.at[sidx_b], sems).wait()

            start_fire(0, rows0, ax0, semr0, sema0)

            def fire(b, _):
                nb = b + 1

                @pl.when(b % 2 == 0)
                def _():
                    @pl.when(nb < NF)
                    def _():
                        @pl.when(b >= 1)
                        def _():
                            wait_scatter(rows1, sidx1, sems1)
                        start_fire(nb, rows1, ax1, semr1, sema1)
                    finish_fire(b, rows0, ax0, sidx0, semr0, sema0, sems0)

                @pl.when(b % 2 == 1)
                def _():
                    @pl.when(nb < NF)
                    def _():
                        wait_scatter(rows0, sidx0, sems0)
                        start_fire(nb, rows0, ax0, semr0, sema0)
                    finish_fire(b, rows1, ax1, sidx1, semr1, sema1, sems1)
                return 0

            lax.fori_loop(0, NF, fire, 0)
            wait_scatter(rows1, sidx1, sems1)
            wait_scatter(rows0, sidx0, sems0)
            return 0

        lax.fori_loop(0, NSB, sb_body, 0)
        plsc.subcore_barrier()
        pltpu.sync_copy(accum.at[pl.ds(sid * RSUB, RSUB)],
                        out.at[pl.ds(lo + sid * RSUB, RSUB)])
        plsc.subcore_barrier()


@functools.cache
def _mp_call():
    return pl.kernel(
        _mp_body,
        out_type=jax.ShapeDtypeStruct((NPAD, H), jnp.float32),
        mesh=plsc.VectorSubcoreMesh(core_axis_name="c", subcore_axis_name="s",
                                    num_cores=2, num_subcores=16),
        scratch_types=[
            pltpu.VMEM_SHARED((RCHUNK + 8, H), jnp.float32),
            pltpu.VMEM((H,), jnp.float32),
            pltpu.VMEM((SB,), jnp.int32),
            pltpu.VMEM((SB,), jnp.int32),
            pltpu.VMEM((BE * 16,), jnp.float32),
            pltpu.VMEM((BE * 16,), jnp.float32),
            pltpu.VMEM((BE,), jnp.int32),
            pltpu.VMEM((BE,), jnp.int32),
            pltpu.VMEM((BE, H), jnp.float32),
            pltpu.VMEM((BE, H), jnp.float32),
            pltpu.VMEM((ZR, H), jnp.float32),
            pltpu.SemaphoreType.DMA,
            pltpu.SemaphoreType.DMA,
            pltpu.SemaphoreType.DMA,
            pltpu.SemaphoreType.DMA,
            pltpu.SemaphoreType.DMA,
            pltpu.SemaphoreType.DMA,
        ],
    )


def _message_pass(hg, src, dst, attrx, wvec):
    return _mp_call()(hg, src, dst, attrx, wvec)


def _mlp_body(h_ref, aggr_ref, w1_ref, b1_ref, w2_ref, b2_ref, eps_ref,
              z_ref, mom_ref):
    i = pl.program_id(0)
    u = (1.0 + eps_ref[0]) * h_ref[...] + aggr_ref[...]
    t = jnp.maximum(jnp.dot(u, w1_ref[...], preferred_element_type=jnp.float32)
                    + b1_ref[...], 0.0)
    z = jnp.dot(t, w2_ref[...], preferred_element_type=jnp.float32) + b2_ref[...]
    z_ref[...] = z
    mom = jnp.stack([jnp.sum(z, axis=0), jnp.sum(z * z, axis=0)])

    @pl.when(i == 0)
    def _():
        mom_ref[...] = jnp.zeros_like(mom_ref)

    mom_ref[...] += mom


def _mlp(h, aggr, W1, b1, W2, b2, eps):
    return pl.pallas_call(
        _mlp_body,
        grid=(NBLK,),
        in_specs=[
            pl.BlockSpec((BN_ROWS, H), lambda i: (i, 0)),
            pl.BlockSpec((BN_ROWS, H), lambda i: (i, 0)),
            pl.BlockSpec((H, H), lambda i: (0, 0)),
            pl.BlockSpec((1, H), lambda i: (0, 0)),
            pl.BlockSpec((H, H), lambda i: (0, 0)),
            pl.BlockSpec((1, H), lambda i: (0, 0)),
            pl.BlockSpec(memory_space=pltpu.SMEM),
        ],
        out_specs=[
            pl.BlockSpec((BN_ROWS, H), lambda i: (i, 0)),
            pl.BlockSpec((2, H), lambda i: (0, 0)),
        ],
        out_shape=[
            jax.ShapeDtypeStruct((N, H), jnp.float32),
            jax.ShapeDtypeStruct((2, H), jnp.float32),
        ],
    )(h, aggr, W1, b1.reshape(1, H), W2, b2.reshape(1, H),
      eps.reshape(1))


def _bnrelu_body(z_ref, mom_ref, g_ref, bt_ref, be_ref, out_ref, outg_ref):
    m = mom_ref[0:1, :] / N
    var = mom_ref[1:2, :] / N - m * m
    inv = jax.lax.rsqrt(var + 1e-5)
    scale = g_ref[...] * inv
    shift = bt_ref[...] - m * scale
    h = jnp.maximum(z_ref[...] * scale + shift, 0.0)
    out_ref[...] = h
    outg_ref[...] = h + be_ref[...]


def _bnrelu(z, mom, g, bt, be):
    return pl.pallas_call(
        _bnrelu_body,
        grid=(NBLK,),
        in_specs=[
            pl.BlockSpec((BN_ROWS, H), lambda i: (i, 0)),
            pl.BlockSpec((2, H), lambda i: (0, 0)),
            pl.BlockSpec((1, H), lambda i: (0, 0)),
            pl.BlockSpec((1, H), lambda i: (0, 0)),
            pl.BlockSpec((1, H), lambda i: (0, 0)),
        ],
        out_specs=[
            pl.BlockSpec((BN_ROWS, H), lambda i: (i, 0)),
            pl.BlockSpec((BN_ROWS, H), lambda i: (i, 0)),
        ],
        out_shape=[
            jax.ShapeDtypeStruct((N, H), jnp.float32),
            jax.ShapeDtypeStruct((N, H), jnp.float32),
        ],
    )(z, mom, g.reshape(1, H), bt.reshape(1, H), be.reshape(1, H))


def _bnrelu_last_body(z_ref, mom_ref, g_ref, bt_ref, out_ref):
    m = mom_ref[0:1, :] / N
    var = mom_ref[1:2, :] / N - m * m
    inv = jax.lax.rsqrt(var + 1e-5)
    scale = g_ref[...] * inv
    shift = bt_ref[...] - m * scale
    out_ref[...] = jnp.maximum(z_ref[...] * scale + shift, 0.0)


def _bnrelu_last(z, mom, g, bt):
    return pl.pallas_call(
        _bnrelu_last_body,
        grid=(NBLK,),
        in_specs=[
            pl.BlockSpec((BN_ROWS, H), lambda i: (i, 0)),
            pl.BlockSpec((2, H), lambda i: (0, 0)),
            pl.BlockSpec((1, H), lambda i: (0, 0)),
            pl.BlockSpec((1, H), lambda i: (0, 0)),
        ],
        out_specs=pl.BlockSpec((BN_ROWS, H), lambda i: (i, 0)),
        out_shape=jax.ShapeDtypeStruct((N, H), jnp.float32),
    )(z, mom, g.reshape(1, H), bt.reshape(1, H))


def _pool_head_body(h_ref, batch_ref, wc1_ref, bc1_ref, wc2_ref, bc2_ref,
                    out_ref, sums_ref, cnt_ref):
    i = pl.program_id(0)

    @pl.when(i == 0)
    def _():
        sums_ref[...] = jnp.zeros_like(sums_ref)
        cnt_ref[...] = jnp.zeros_like(cnt_ref)

    b = batch_ref[0, 0, :].reshape(BN_ROWS, 1)
    onehot = (b == jax.lax.broadcasted_iota(jnp.int32, (1, G), 1)
              ).astype(jnp.float32)
    sums_ref[...] += jax.lax.dot_general(
        onehot, h_ref[...], (((0,), (0,)), ((), ())),
        preferred_element_type=jnp.float32)
    cnt_ref[...] += jnp.sum(onehot, axis=0, keepdims=True)

    @pl.when(i == NBLK - 1)
    def _():
        pooled = sums_ref[...] / jnp.maximum(cnt_ref[...], 1.0).reshape(G, 1)
        t = jnp.maximum(
            jnp.dot(pooled, wc1_ref[...], preferred_element_type=jnp.float32)
            + bc1_ref[...], 0.0)
        o = jnp.dot(t, wc2_ref[...], preferred_element_type=jnp.float32) \
            + bc2_ref[...]
        out_ref[...] = jax.nn.sigmoid(o).reshape(1, G)


def _pool_head(h, batch, Wc1, bc1, Wc2, bc2):
    out, _, _ = pl.pallas_call(
        _pool_head_body,
        grid=(NBLK,),
        in_specs=[
            pl.BlockSpec((BN_ROWS, H), lambda i: (i, 0)),
            pl.BlockSpec((1, 1, BN_ROWS), lambda i: (i, 0, 0)),
            pl.BlockSpec((H, H // 2), lambda i: (0, 0)),
            pl.BlockSpec((1, H // 2), lambda i: (0, 0)),
            pl.BlockSpec((H // 2, 1), lambda i: (0, 0)),
            pl.BlockSpec((1, 1), lambda i: (0, 0)),
        ],
        out_specs=[
            pl.BlockSpec((1, G), lambda i: (0, 0)),
            pl.BlockSpec((G, H), lambda i: (0, 0)),
            pl.BlockSpec((1, G), lambda i: (0, 0)),
        ],
        out_shape=[
            jax.ShapeDtypeStruct((1, G), jnp.float32),
            jax.ShapeDtypeStruct((G, H), jnp.float32),
            jax.ShapeDtypeStruct((1, G), jnp.float32),
        ],
    )(h, batch.reshape(NBLK, 1, BN_ROWS), Wc1, bc1.reshape(1, H // 2),
      Wc2, bc2.reshape(1, 1))
    return out.reshape(G)


def kernel(x, edge_index, edge_attr, batch, Wn, bn, We, be, W1_0, b1_0, W2_0,
           b2_0, eps_0, g_0, bt_0, W1_1, b1_1, W2_1, b2_1, eps_1, g_1, bt_1,
           Wc1, bc1, Wc2, bc2):
    src, dst = edge_index[0], edge_index[1]
    attrx = _attr_expand(edge_attr.reshape(E))
    wvec = We.reshape(H)
    h, hg = _embed(x, Wn, bn, be)
    layers = ((W1_0, b1_0, W2_0, b2_0, eps_0, g_0, bt_0),
              (W1_1, b1_1, W2_1, b2_1, eps_1, g_1, bt_1))
    for li, (W1, b1, W2, b2, eps, g, bt) in enumerate(layers):
        aggr = _message_pass(hg, src, dst, attrx, wvec)
        z, mom = _mlp(h, aggr, W1, b1, W2, b2, eps)
        if li == 0:
            h, hg = _bnrelu(z, mom, g, bt, be)
        else:
            h = _bnrelu_last(z, mom, g, bt)
    return _pool_head(h, batch, Wc1, bc1, Wc2, bc2)
